# baseline (speedup 1.0000x reference)
import functools

import jax
import jax.numpy as jnp
from jax import lax
from jax.experimental import pallas as pl
from jax.experimental.pallas import tpu as pltpu
from jax.experimental.pallas import tpu_sc as plsc

_B, _H, _W, _C = 4, 384, 384, 96
_HW = _H * _W
_NW = 32
_PPT = _HW // _NW
_CH = 32
_NCHUNK = _PPT // _CH
_NL = 16


def _warp_body(img_hbm, trf_hbm, out_hbm, tx_v, ty_v,
               idx_v, w_v, g_v, o_v, gsem, osem):
    wid = lax.axis_index("s") * 2 + lax.axis_index("c")
    base = wid * _PPT

    pltpu.sync_copy(trf_hbm.at[pl.ds(base, _PPT)], tx_v)
    pltpu.sync_copy(trf_hbm.at[pl.ds(_HW + base, _PPT)], ty_v)

    fone = jnp.float32(1.0)

    def compute_idx(gg, slot):
        off = gg * _CH
        for h in range(_CH // _NL):
            s16 = off + h * _NL
            p = base + s16 + lax.iota(jnp.int32, _NL)
            i = lax.div(p, _W)
            j = p - i * _W

            tx = tx_v[pl.ds(s16, _NL)]
            ty = ty_v[pl.ds(s16, _NL)]

            locx = jnp.clip(i.astype(jnp.float32) + tx, 0.0, float(_H - 1))
            x0 = jnp.minimum(locx.astype(jnp.int32), _H - 2)
            fx = locx - x0.astype(jnp.float32)

            locy = jnp.clip(j.astype(jnp.float32) + ty, 0.0, float(_W - 1))
            y0 = jnp.minimum(locy.astype(jnp.int32), _W - 2)
            fy = locy - y0.astype(jnp.float32)

            i00 = x0 * _W + y0
            gx = fone - fx
            gy = fone - fy

            idx_v[slot, pl.ds(0 * _CH + h * _NL, _NL)] = i00
            idx_v[slot, pl.ds(1 * _CH + h * _NL, _NL)] = i00 + 1
            idx_v[slot, pl.ds(2 * _CH + h * _NL, _NL)] = i00 + _W
            idx_v[slot, pl.ds(3 * _CH + h * _NL, _NL)] = i00 + _W + 1
            w_v[slot, pl.ds(0 * _CH + h * _NL, _NL)] = gx * gy
            w_v[slot, pl.ds(1 * _CH + h * _NL, _NL)] = gx * fy
            w_v[slot, pl.ds(2 * _CH + h * _NL, _NL)] = fx * gy
            w_v[slot, pl.ds(3 * _CH + h * _NL, _NL)] = fx * fy

    def start_gather(slot):
        pltpu.async_copy(img_hbm.at[idx_v.at[slot]], g_v.at[slot],
                         gsem.at[slot])

    def wait_gather(slot):
        pltpu.make_async_copy(img_hbm.at[idx_v.at[slot]], g_v.at[slot],
                              gsem.at[slot]).wait()

    def blend(slot):
        def px_body(pp, c2):
            w00 = w_v[slot, pl.ds(0 * _CH + pp, _NL)][0]
            w01 = w_v[slot, pl.ds(1 * _CH + pp, _NL)][0]
            w10 = w_v[slot, pl.ds(2 * _CH + pp, _NL)][0]
            w11 = w_v[slot, pl.ds(3 * _CH + pp, _NL)][0]
            for c in range(_C // _NL):
                sl = pl.ds(c * _NL, _NL)
                o_v[slot, pl.ds(pp * _C + c * _NL, _NL)] = (
                    w00 * g_v[slot, 0 * _CH + pp, sl]
                    + w01 * g_v[slot, 1 * _CH + pp, sl]
                    + w10 * g_v[slot, 2 * _CH + pp, sl]
                    + w11 * g_v[slot, 3 * _CH + pp, sl])
            return c2

        lax.fori_loop(0, _CH, px_body, 0, unroll=False)

    def start_out(slot, gg):
        pltpu.async_copy(o_v.at[slot],
                         out_hbm.at[pl.ds((base + gg * _CH) * _C, _CH * _C)],
                         osem.at[slot])

    def wait_out(slot, gg):
        pltpu.make_async_copy(o_v.at[slot],
                              out_hbm.at[pl.ds((base + gg * _CH) * _C,
                                               _CH * _C)],
                              osem.at[slot]).wait()

    compute_idx(0, 0)
    start_gather(0)

    def body(g, carry):
        slot = g & 1
        nslot = 1 - slot

        @pl.when(g + 1 < _NCHUNK)
        def _():
            compute_idx(g + 1, nslot)
            start_gather(nslot)

        wait_gather(slot)

        @pl.when(g >= 2)
        def _():
            wait_out(slot, g - 2)

        blend(slot)
        start_out(slot, g)
        return carry

    lax.fori_loop(0, _NCHUNK, body, 0, unroll=False)

    wait_out((_NCHUNK - 2) & 1, _NCHUNK - 2)
    wait_out((_NCHUNK - 1) & 1, _NCHUNK - 1)


@jax.jit
def _warp_sc(img_flat, txy):
    mesh = plsc.VectorSubcoreMesh(core_axis_name="c", subcore_axis_name="s")
    return pl.kernel(
        _warp_body,
        out_type=jax.ShapeDtypeStruct((_HW * _C,), jnp.float32),
        mesh=mesh,
        scratch_types=[
            pltpu.VMEM((_PPT,), jnp.float32),
            pltpu.VMEM((_PPT,), jnp.float32),
            pltpu.VMEM((2, 4 * _CH), jnp.int32),
            pltpu.VMEM((2, 4 * _CH + _NL), jnp.float32),
            pltpu.VMEM((2, 4 * _CH, _C), jnp.float32),
            pltpu.VMEM((2, _CH * _C), jnp.float32),
            pltpu.SemaphoreType.DMA((2,)),
            pltpu.SemaphoreType.DMA((2,)),
        ],
        compiler_params=pltpu.CompilerParams(use_tc_tiling_on_sc=False),
    )(img_flat, txy)


def kernel(img, trf):
    B, H, W, C = img.shape
    img5 = img.reshape(B, H * W, C)
    trf5 = trf.reshape(B, H * W, 2)
    outs = []
    for b in range(B):
        txy_b = trf5[b].T.reshape(-1)
        outs.append(_warp_sc(img5[b], txy_b))
    return jnp.concatenate(outs).reshape(B, H, W, C)


# speedup vs baseline: 1.0480x; 1.0480x over previous
import functools

import jax
import jax.numpy as jnp
from jax import lax
from jax.experimental import pallas as pl
from jax.experimental.pallas import tpu as pltpu
from jax.experimental.pallas import tpu_sc as plsc

_B, _H, _W, _C = 4, 384, 384, 96
_HW = _H * _W
_NW = 32
_PPT = _HW // _NW
_CH = 64
_NCHUNK = _PPT // _CH
_NL = 16


def _warp_body(img_hbm, trf_hbm, out_hbm, tx_v, ty_v,
               idx_v, w_v, g_v, o_v, gsem, osem):
    wid = lax.axis_index("s") * 2 + lax.axis_index("c")
    base = wid * _PPT

    pltpu.sync_copy(trf_hbm.at[pl.ds(base, _PPT)], tx_v)
    pltpu.sync_copy(trf_hbm.at[pl.ds(_HW + base, _PPT)], ty_v)

    fone = jnp.float32(1.0)

    def compute_idx(gg, slot):
        off = gg * _CH
        for h in range(_CH // _NL):
            s16 = off + h * _NL
            p = base + s16 + lax.iota(jnp.int32, _NL)
            i = lax.div(p, _W)
            j = p - i * _W

            tx = tx_v[pl.ds(s16, _NL)]
            ty = ty_v[pl.ds(s16, _NL)]

            locx = jnp.clip(i.astype(jnp.float32) + tx, 0.0, float(_H - 1))
            x0 = jnp.minimum(locx.astype(jnp.int32), _H - 2)
            fx = locx - x0.astype(jnp.float32)

            locy = jnp.clip(j.astype(jnp.float32) + ty, 0.0, float(_W - 1))
            y0 = jnp.minimum(locy.astype(jnp.int32), _W - 2)
            fy = locy - y0.astype(jnp.float32)

            i00 = x0 * _W + y0
            gx = fone - fx
            gy = fone - fy

            sl = pl.ds(h * _NL, _NL)
            idx_v[slot, 0, sl] = i00
            idx_v[slot, 1, sl] = i00 + 1
            idx_v[slot, 2, sl] = i00 + _W
            idx_v[slot, 3, sl] = i00 + _W + 1
            w_v[slot, 0, sl] = gx * gy
            w_v[slot, 1, sl] = gx * fy
            w_v[slot, 2, sl] = fx * gy
            w_v[slot, 3, sl] = fx * fy

    def start_gather(slot):
        for n in range(4):
            pltpu.async_copy(img_hbm.at[idx_v.at[slot, n]], g_v.at[slot, n],
                             gsem.at[slot])

    def wait_gather(slot):
        pltpu.make_async_copy(img_hbm.at[idx_v.at[slot, 0]], g_v.at[slot],
                              gsem.at[slot]).wait()

    def blend(slot):
        def px_body(pp, c2):
            w00 = w_v[slot, 0, pl.ds(pp, _NL)][0]
            w01 = w_v[slot, 1, pl.ds(pp, _NL)][0]
            w10 = w_v[slot, 2, pl.ds(pp, _NL)][0]
            w11 = w_v[slot, 3, pl.ds(pp, _NL)][0]
            for c in range(_C // _NL):
                sl = pl.ds(c * _NL, _NL)
                o_v[slot, pp, sl] = (w00 * g_v[slot, 0, pp, sl]
                                     + w01 * g_v[slot, 1, pp, sl]
                                     + w10 * g_v[slot, 2, pp, sl]
                                     + w11 * g_v[slot, 3, pp, sl])
            return c2

        lax.fori_loop(0, _CH, px_body, 0, unroll=False)

    def start_out(slot, gg):
        pltpu.async_copy(o_v.at[slot], out_hbm.at[pl.ds(base + gg * _CH, _CH)],
                         osem.at[slot])

    def wait_out(slot, gg):
        pltpu.make_async_copy(o_v.at[slot],
                              out_hbm.at[pl.ds(base + gg * _CH, _CH)],
                              osem.at[slot]).wait()

    compute_idx(0, 0)
    start_gather(0)

    def body(g, carry):
        slot = g & 1
        nslot = 1 - slot

        @pl.when(g + 1 < _NCHUNK)
        def _():
            compute_idx(g + 1, nslot)
            start_gather(nslot)

        wait_gather(slot)

        @pl.when(g >= 2)
        def _():
            wait_out(slot, g - 2)

        blend(slot)
        start_out(slot, g)
        return carry

    lax.fori_loop(0, _NCHUNK, body, 0, unroll=False)

    wait_out((_NCHUNK - 2) & 1, _NCHUNK - 2)
    wait_out((_NCHUNK - 1) & 1, _NCHUNK - 1)


@jax.jit
def _warp_sc(img_flat, txy):
    mesh = plsc.VectorSubcoreMesh(core_axis_name="c", subcore_axis_name="s")
    return pl.kernel(
        _warp_body,
        out_type=jax.ShapeDtypeStruct((_HW, _C), jnp.float32),
        mesh=mesh,
        scratch_types=[
            pltpu.VMEM((_PPT,), jnp.float32),
            pltpu.VMEM((_PPT,), jnp.float32),
            pltpu.VMEM((2, 4, _CH), jnp.int32),
            pltpu.VMEM((2, 4, _CH + _NL), jnp.float32),
            pltpu.VMEM((2, 4, _CH, _C), jnp.float32),
            pltpu.VMEM((2, _CH, _C), jnp.float32),
            pltpu.SemaphoreType.DMA((2,)),
            pltpu.SemaphoreType.DMA((2,)),
        ],
        compiler_params=pltpu.CompilerParams(use_tc_tiling_on_sc=False),
    )(img_flat, txy)


def kernel(img, trf):
    B, H, W, C = img.shape
    outs = []
    for b in range(B):
        img_b = img[b].reshape(H * W, C)
        txy_b = trf[b].reshape(H * W, 2).T.reshape(-1)
        outs.append(_warp_sc(img_b, txy_b))
    return jnp.stack(outs).reshape(B, H, W, C)


# speedup vs baseline: 1.0803x; 1.0308x over previous
import functools

import jax
import jax.numpy as jnp
from jax import lax
from jax.experimental import pallas as pl
from jax.experimental.pallas import tpu as pltpu
from jax.experimental.pallas import tpu_sc as plsc

_B, _H, _W, _C = 4, 384, 384, 96
_HW = _H * _W
_NW = 32
_PPT = _HW // _NW
_CH = 32
_NCHUNK = _PPT // _CH
_NL = 16


def _warp_body(img_hbm, trf_hbm, out_hbm, tx_v, ty_v,
               idx_v, w_v, g_v, o_v, gsem, osem):
    wid = lax.axis_index("s") * 2 + lax.axis_index("c")
    base = wid * _PPT

    pltpu.sync_copy(trf_hbm.at[0, pl.ds(base, _PPT)], tx_v)
    pltpu.sync_copy(trf_hbm.at[1, pl.ds(base, _PPT)], ty_v)

    fone = jnp.float32(1.0)

    def compute_idx(gg, slot):
        off = gg * _CH
        for h in range(_CH // _NL):
            s16 = off + h * _NL
            p = base + s16 + lax.iota(jnp.int32, _NL)
            i = lax.div(p, _W)
            j = p - i * _W

            tx = tx_v[pl.ds(s16, _NL)]
            ty = ty_v[pl.ds(s16, _NL)]

            locx = jnp.clip(i.astype(jnp.float32) + tx, 0.0, float(_H - 1))
            x0 = jnp.minimum(locx.astype(jnp.int32), _H - 2)
            fx = locx - x0.astype(jnp.float32)

            locy = jnp.clip(j.astype(jnp.float32) + ty, 0.0, float(_W - 1))
            y0 = jnp.minimum(locy.astype(jnp.int32), _W - 2)
            fy = locy - y0.astype(jnp.float32)

            i00 = x0 * _W + y0
            gx = fone - fx
            gy = fone - fy

            idx_v[slot, pl.ds(0 * _CH + h * _NL, _NL)] = i00
            idx_v[slot, pl.ds(1 * _CH + h * _NL, _NL)] = i00 + 1
            idx_v[slot, pl.ds(2 * _CH + h * _NL, _NL)] = i00 + _W
            idx_v[slot, pl.ds(3 * _CH + h * _NL, _NL)] = i00 + _W + 1
            w_v[slot, pl.ds(0 * _CH + h * _NL, _NL)] = gx * gy
            w_v[slot, pl.ds(1 * _CH + h * _NL, _NL)] = gx * fy
            w_v[slot, pl.ds(2 * _CH + h * _NL, _NL)] = fx * gy
            w_v[slot, pl.ds(3 * _CH + h * _NL, _NL)] = fx * fy

    def start_gather(slot):
        pltpu.async_copy(img_hbm.at[idx_v.at[slot]], g_v.at[slot],
                         gsem.at[slot])

    def wait_gather(slot):
        pltpu.make_async_copy(img_hbm.at[idx_v.at[slot]], g_v.at[slot],
                              gsem.at[slot]).wait()

    def blend(slot):
        def px_body(pp, c2):
            w00 = w_v[slot, pl.ds(0 * _CH + pp, _NL)][0]
            w01 = w_v[slot, pl.ds(1 * _CH + pp, _NL)][0]
            w10 = w_v[slot, pl.ds(2 * _CH + pp, _NL)][0]
            w11 = w_v[slot, pl.ds(3 * _CH + pp, _NL)][0]
            for c in range(_C // _NL):
                sl = pl.ds(c * _NL, _NL)
                o_v[slot, pp, sl] = (w00 * g_v[slot, 0 * _CH + pp, sl]
                                     + w01 * g_v[slot, 1 * _CH + pp, sl]
                                     + w10 * g_v[slot, 2 * _CH + pp, sl]
                                     + w11 * g_v[slot, 3 * _CH + pp, sl])
            return c2

        lax.fori_loop(0, _CH, px_body, 0, unroll=False)

    def start_out(slot, gg):
        pltpu.async_copy(o_v.at[slot], out_hbm.at[pl.ds(base + gg * _CH, _CH)],
                         osem.at[slot])

    def wait_out(slot, gg):
        pltpu.make_async_copy(o_v.at[slot],
                              out_hbm.at[pl.ds(base + gg * _CH, _CH)],
                              osem.at[slot]).wait()

    compute_idx(0, 0)
    start_gather(0)

    def body(g, carry):
        slot = g & 1
        nslot = 1 - slot

        @pl.when(g + 1 < _NCHUNK)
        def _():
            compute_idx(g + 1, nslot)
            start_gather(nslot)

        wait_gather(slot)

        @pl.when(g >= 2)
        def _():
            wait_out(slot, g - 2)

        blend(slot)
        start_out(slot, g)
        return carry

    lax.fori_loop(0, _NCHUNK, body, 0, unroll=False)

    wait_out((_NCHUNK - 2) & 1, _NCHUNK - 2)
    wait_out((_NCHUNK - 1) & 1, _NCHUNK - 1)


@jax.jit
def _warp_sc(img_flat, txy):
    mesh = plsc.VectorSubcoreMesh(core_axis_name="c", subcore_axis_name="s")
    return pl.kernel(
        _warp_body,
        out_type=jax.ShapeDtypeStruct((_HW, _C), jnp.float32),
        mesh=mesh,
        scratch_types=[
            pltpu.VMEM((_PPT,), jnp.float32),
            pltpu.VMEM((_PPT,), jnp.float32),
            pltpu.VMEM((2, 4 * _CH), jnp.int32),
            pltpu.VMEM((2, 4 * _CH + _NL), jnp.float32),
            pltpu.VMEM((2, 4 * _CH, _C), jnp.float32),
            pltpu.VMEM((2, _CH, _C), jnp.float32),
            pltpu.SemaphoreType.DMA((2,)),
            pltpu.SemaphoreType.DMA((2,)),
        ],
        compiler_params=pltpu.CompilerParams(use_tc_tiling_on_sc=False),
    )(img_flat, txy)


def kernel(img, trf):
    B, H, W, C = img.shape
    outs = []
    for b in range(B):
        img_b = img[b].reshape(H * W, C)
        txy_b = trf[b].reshape(H * W, 2).T
        outs.append(_warp_sc(img_b, txy_b))
    return jnp.stack(outs).reshape(B, H, W, C)
